# baseline (speedup 1.0000x reference)
import functools

import jax
import jax.numpy as jnp
from jax import lax
from jax.experimental import pallas as pl
from jax.experimental.pallas import tpu as pltpu
from jax.experimental.pallas import tpu_sc as plsc

N = 10002
D = 128
HALF = 64
N_PAD = 10240
RPT = 640
C = 128
NTILES = 16
NCORES = 2
VPR = HALF // 16
DEPTH = 4


def _scale_chunk(gbuf, vb):
    def group(g, carry):
        vv = vb[pl.ds(g * 16, 16)]
        for l in range(16):
            e = g * 16 + l
            sval = vv[l]
            for j in range(VPR):
                sl = pl.ds(j * 16, 16)
                gbuf[e, sl] = gbuf[e, sl] * sval
        return carry
    lax.fori_loop(0, C // 16, group, 0)


def _body(final, nchunks, *refs):
    if final:
        (x_hbm, cols_hbm, rows_hbm, vals_hbm, x0_hbm, x1_hbm, out_hbm,
         acc, colsb, rowsb, *rest) = refs
    else:
        (x_hbm, cols_hbm, rows_hbm, vals_hbm, out_hbm,
         acc, colsb, rowsb, *rest) = refs
    vb = rest[0:DEPTH]
    gb = rest[DEPTH:2 * DEPTH]
    sem_g = rest[2 * DEPTH:3 * DEPTH]
    sem_v = rest[3 * DEPTH:4 * DEPTH]
    sem_s = rest[4 * DEPTH:4 * DEPTH + 2]

    c = lax.axis_index("c")
    s = lax.axis_index("s")

    def zrow(i, carry):
        for j in range(VPR):
            gb[0][i, pl.ds(j * 16, 16)] = jnp.zeros((16,), jnp.float32)
        return carry
    lax.fori_loop(0, C, zrow, 0)
    rbase = s * RPT
    for b in range(RPT // C):
        pltpu.sync_copy(gb[0], acc.at[pl.ds(rbase + b * C, C)])

    crow0 = (c * NTILES + s) * nchunks
    erow0 = s * nchunks
    pltpu.sync_copy(cols_hbm.at[pl.ds(crow0, nchunks)], colsb)
    pltpu.sync_copy(rows_hbm.at[pl.ds(erow0, nchunks)], rowsb)
    plsc.subcore_barrier()

    def issue_gather(k, slot):
        pltpu.async_copy(x_hbm.at[colsb.at[k]], gb[slot], sem_g[slot])
        pltpu.async_copy(vals_hbm.at[erow0 + k], vb[slot], sem_v[slot])

    def wait_gather(k, slot):
        pltpu.make_async_copy(x_hbm.at[colsb.at[k]], gb[slot],
                              sem_g[slot]).wait()
        pltpu.make_async_copy(vals_hbm.at[erow0 + k], vb[slot],
                              sem_v[slot]).wait()

    def wait_scatter(k, slot, parity):
        pltpu.make_async_copy(gb[slot], acc.at[rowsb.at[k]],
                              sem_s[parity]).wait()

    niter = nchunks // DEPTH
    issue_gather(0, 0)
    issue_gather(1, 1)

    def iter_body(i, carry):
        for j in range(DEPTH):
            k = i * DEPTH + j
            nslot = (j + 2) % DEPTH
            parity = j % 2
            wait_gather(k, j)
            _scale_chunk(gb[j], vb[j])
            if j < 2:
                @pl.when(i > 0)
                def _():
                    wait_scatter(k - 2, nslot, parity)
            else:
                wait_scatter(k - 2, nslot, parity)
            pltpu.async_copy(gb[j], acc.at[rowsb.at[k]], sem_s[parity],
                             add=True)
            if j < 2:
                issue_gather(k + 2, nslot)
            else:
                @pl.when(i < niter - 1)
                def _():
                    issue_gather(k + 2, nslot)
        return carry
    lax.fori_loop(0, niter, iter_body, 0)
    wait_scatter(nchunks - 2, 2, 0)
    wait_scatter(nchunks - 1, 3, 1)

    plsc.subcore_barrier()

    obase = c * N_PAD + rbase
    for b in range(RPT // C):
        r0 = rbase + b * C
        o0 = obase + b * C
        if not final:
            pltpu.sync_copy(acc.at[pl.ds(r0, C)], out_hbm.at[pl.ds(o0, C)])
        else:
            pltpu.sync_copy(acc.at[pl.ds(r0, C)], gb[0])
            pltpu.sync_copy(x0_hbm.at[pl.ds(o0, C)], gb[1])
            pltpu.sync_copy(x1_hbm.at[pl.ds(o0, C)], gb[2])

            def crow(i, carry):
                for j in range(VPR):
                    sl = pl.ds(j * 16, 16)
                    gb[0][i, sl] = (
                        gb[0][i, sl] + gb[1][i, sl] + gb[2][i, sl]
                    ) * (1.0 / 3.0)
                return carry
            lax.fori_loop(0, C, crow, 0)
            pltpu.sync_copy(gb[0], out_hbm.at[pl.ds(o0, C)])


def _make_kernel(nchunks, final):
    mesh = plsc.VectorSubcoreMesh(core_axis_name="c", subcore_axis_name="s")
    scratch = [
        pltpu.VMEM_SHARED((N_PAD, HALF), jnp.float32),
        pltpu.VMEM((nchunks, C), jnp.int32),
        pltpu.VMEM((nchunks, C), jnp.int32),
    ]
    scratch += [pltpu.VMEM((C,), jnp.float32) for _ in range(DEPTH)]
    scratch += [pltpu.VMEM((C, HALF), jnp.float32) for _ in range(DEPTH)]
    scratch += [pltpu.SemaphoreType.DMA for _ in range(DEPTH)]
    scratch += [pltpu.SemaphoreType.DMA for _ in range(DEPTH)]
    scratch += [pltpu.SemaphoreType.DMA for _ in range(2)]
    return pl.kernel(
        functools.partial(_body, final, nchunks),
        out_type=jax.ShapeDtypeStruct((2 * N_PAD, HALF), jnp.float32),
        mesh=mesh,
        scratch_types=scratch,
        compiler_params=pltpu.CompilerParams(use_tc_tiling_on_sc=False),
    )


def kernel(adjacency_indices, adjacency_values, embedding):
    rows = adjacency_indices[0].astype(jnp.int32)
    cols = adjacency_indices[1].astype(jnp.int32)
    vals = adjacency_values.astype(jnp.float32)
    e = vals.shape[0]
    ept = -(-(e // NTILES) // (DEPTH * C)) * (DEPTH * C)
    nchunks = ept // C
    e_pad = ept * NTILES

    cols_p = jnp.pad(cols, (0, e_pad - e))
    rows_p = jnp.pad(rows, (0, e_pad - e), constant_values=N)
    vals_p = jnp.pad(vals, (0, e_pad - e))
    cols2 = jnp.concatenate([cols_p, cols_p + N_PAD]).reshape(-1, C)
    rows2 = rows_p.reshape(-1, C)
    vals2 = vals_p.reshape(-1, C)

    emb_pad = jnp.pad(embedding.astype(jnp.float32),
                      ((0, N_PAD - N), (0, 0)))
    x0s = jnp.concatenate([emb_pad[:, :HALF], emb_pad[:, HALF:]], axis=0)

    layer_k = _make_kernel(nchunks, final=False)
    final_k = _make_kernel(nchunks, final=True)

    x1s = layer_k(x0s, cols2, rows2, vals2)
    outs = final_k(x1s, cols2, rows2, vals2, x0s, x1s)

    full = jnp.concatenate([outs[:N], outs[N_PAD:N_PAD + N]], axis=1)
    ds3 = N // 3
    return jnp.concatenate(
        [full[:ds3], full[ds3:2 * ds3], full[2 * ds3:]], axis=1)


# speedup vs baseline: 1.0014x; 1.0014x over previous
import functools

import jax
import jax.numpy as jnp
from jax import lax
from jax.experimental import pallas as pl
from jax.experimental.pallas import tpu as pltpu
from jax.experimental.pallas import tpu_sc as plsc

N = 10002
D = 128
HALF = 64
N_PAD = 10240
RPT = 640
C = 128
NTILES = 16
NCORES = 2
VPR = HALF // 16
DEPTH = 4
RDEPTH = 8
UNROLL = 8


def _scale_chunk(gbuf, vb):
    def group(g, carry):
        vv = vb[pl.ds(g * 16, 16)]
        for l in range(16):
            e = g * 16 + l
            sval = vv[l]
            for j in range(VPR):
                sl = pl.ds(j * 16, 16)
                gbuf[e, sl] = gbuf[e, sl] * sval
        return carry
    lax.fori_loop(0, C // 16, group, 0)


def _body(final, nchunks, *refs):
    if final:
        (x_hbm, cols_hbm, rows_hbm, vals_hbm, x0_hbm, x1_hbm, out_hbm,
         acc, *rest) = refs
    else:
        (x_hbm, cols_hbm, rows_hbm, vals_hbm, out_hbm, acc, *rest) = refs
    cv = rest[0:DEPTH]
    vb = rest[DEPTH:2 * DEPTH]
    gb = rest[2 * DEPTH:3 * DEPTH]
    rv = rest[3 * DEPTH:3 * DEPTH + RDEPTH]
    base = 3 * DEPTH + RDEPTH
    sem_i = rest[base:base + DEPTH]
    sem_g = rest[base + DEPTH:base + 2 * DEPTH]
    sem_s = rest[base + 2 * DEPTH:base + 2 * DEPTH + 2]

    c = lax.axis_index("c")
    s = lax.axis_index("s")

    def zrow(i, carry):
        for j in range(VPR):
            gb[0][i, pl.ds(j * 16, 16)] = jnp.zeros((16,), jnp.float32)
        return carry
    lax.fori_loop(0, C, zrow, 0)
    rbase = s * RPT
    for b in range(RPT // C):
        pltpu.sync_copy(gb[0], acc.at[pl.ds(rbase + b * C, C)])
    plsc.subcore_barrier()

    crow0 = (c * NTILES + s) * nchunks
    erow0 = s * nchunks

    def iter_chunks(i, carry):
        for j in range(UNROLL):
            k = i * UNROLL + j
            s4 = j % DEPTH
            s8 = j % RDEPTH
            p = j % 2
            pltpu.make_async_copy(x_hbm.at[cv[s4]], gb[s4],
                                  sem_g[s4]).wait()
            _scale_chunk(gb[s4], vb[s4])
            if j >= 2:
                pltpu.make_async_copy(gb[(j + 2) % DEPTH],
                                      acc.at[rv[(j + 2) % RDEPTH]],
                                      sem_s[p]).wait()
            else:
                @pl.when(i > 0)
                def _():
                    pltpu.make_async_copy(gb[(j + 2) % DEPTH],
                                          acc.at[rv[(j + 2) % RDEPTH]],
                                          sem_s[p]).wait()
            pltpu.async_copy(gb[s4], acc.at[rv[s8]], sem_s[p], add=True)
            i3s4 = (j + 3) % DEPTH
            i3s8 = (j + 3) % RDEPTH

            def idx3():
                pltpu.async_copy(cols_hbm.at[crow0 + k + 3], cv[i3s4],
                                 sem_i[i3s4])
                pltpu.async_copy(rows_hbm.at[erow0 + k + 3], rv[i3s8],
                                 sem_i[i3s4])
                pltpu.async_copy(vals_hbm.at[erow0 + k + 3], vb[i3s4],
                                 sem_i[i3s4])
            if j < UNROLL - 3:
                idx3()
            else:
                @pl.when(i < nchunks // UNROLL - 1)
                def _():
                    idx3()
            g2s4 = (j + 2) % DEPTH

            def gather2():
                pltpu.make_async_copy(cols_hbm.at[crow0 + k + 2], cv[g2s4],
                                      sem_i[g2s4]).wait()
                pltpu.make_async_copy(rows_hbm.at[erow0 + k + 2],
                                      rv[(j + 2) % RDEPTH],
                                      sem_i[g2s4]).wait()
                pltpu.make_async_copy(vals_hbm.at[erow0 + k + 2], vb[g2s4],
                                      sem_i[g2s4]).wait()
                pltpu.async_copy(x_hbm.at[cv[g2s4]], gb[g2s4], sem_g[g2s4])
            if j < UNROLL - 2:
                gather2()
            else:
                @pl.when(i < nchunks // UNROLL - 1)
                def _():
                    gather2()
        return carry

    for k0 in range(3):
        pltpu.async_copy(cols_hbm.at[crow0 + k0], cv[k0], sem_i[k0])
        pltpu.async_copy(rows_hbm.at[erow0 + k0], rv[k0], sem_i[k0])
        pltpu.async_copy(vals_hbm.at[erow0 + k0], vb[k0], sem_i[k0])
    for k0 in range(2):
        pltpu.make_async_copy(cols_hbm.at[crow0 + k0], cv[k0],
                              sem_i[k0]).wait()
        pltpu.make_async_copy(rows_hbm.at[erow0 + k0], rv[k0],
                              sem_i[k0]).wait()
        pltpu.make_async_copy(vals_hbm.at[erow0 + k0], vb[k0],
                              sem_i[k0]).wait()
        pltpu.async_copy(x_hbm.at[cv[k0]], gb[k0], sem_g[k0])

    lax.fori_loop(0, nchunks // UNROLL, iter_chunks, 0)
    pltpu.make_async_copy(gb[(nchunks - 2) % DEPTH],
                          acc.at[rv[(nchunks - 2) % RDEPTH]],
                          sem_s[(nchunks - 2) % 2]).wait()
    pltpu.make_async_copy(gb[(nchunks - 1) % DEPTH],
                          acc.at[rv[(nchunks - 1) % RDEPTH]],
                          sem_s[(nchunks - 1) % 2]).wait()

    plsc.subcore_barrier()

    obase = c * N_PAD + rbase
    for b in range(RPT // C):
        r0 = rbase + b * C
        o0 = obase + b * C
        if not final:
            pltpu.sync_copy(acc.at[pl.ds(r0, C)], out_hbm.at[pl.ds(o0, C)])
        else:
            pltpu.sync_copy(acc.at[pl.ds(r0, C)], gb[0])
            pltpu.sync_copy(x0_hbm.at[pl.ds(o0, C)], gb[1])
            pltpu.sync_copy(x1_hbm.at[pl.ds(o0, C)], gb[2])

            def crow(i, carry):
                for j in range(VPR):
                    sl = pl.ds(j * 16, 16)
                    gb[0][i, sl] = (
                        gb[0][i, sl] + gb[1][i, sl] + gb[2][i, sl]
                    ) * (1.0 / 3.0)
                return carry
            lax.fori_loop(0, C, crow, 0)
            pltpu.sync_copy(gb[0], out_hbm.at[pl.ds(o0, C)])


def _make_kernel(nchunks, final):
    mesh = plsc.VectorSubcoreMesh(core_axis_name="c", subcore_axis_name="s")
    scratch = [
        pltpu.VMEM_SHARED((N_PAD, HALF), jnp.float32),
    ]
    scratch += [pltpu.VMEM((C,), jnp.int32) for _ in range(DEPTH)]
    scratch += [pltpu.VMEM((C,), jnp.float32) for _ in range(DEPTH)]
    scratch += [pltpu.VMEM((C, HALF), jnp.float32) for _ in range(DEPTH)]
    scratch += [pltpu.VMEM((C,), jnp.int32) for _ in range(RDEPTH)]
    scratch += [pltpu.SemaphoreType.DMA for _ in range(DEPTH)]
    scratch += [pltpu.SemaphoreType.DMA for _ in range(DEPTH)]
    scratch += [pltpu.SemaphoreType.DMA for _ in range(2)]
    return pl.kernel(
        functools.partial(_body, final, nchunks),
        out_type=jax.ShapeDtypeStruct((2 * N_PAD, HALF), jnp.float32),
        mesh=mesh,
        scratch_types=scratch,
        compiler_params=pltpu.CompilerParams(use_tc_tiling_on_sc=False),
    )


def kernel(adjacency_indices, adjacency_values, embedding):
    rows = adjacency_indices[0].astype(jnp.int32)
    cols = adjacency_indices[1].astype(jnp.int32)
    vals = adjacency_values.astype(jnp.float32)
    e = vals.shape[0]
    ept = -(-(e // NTILES) // (UNROLL * C)) * (UNROLL * C)
    nchunks = ept // C
    e_pad = ept * NTILES

    cols_p = jnp.pad(cols, (0, e_pad - e))
    rows_p = jnp.pad(rows, (0, e_pad - e), constant_values=N)
    vals_p = jnp.pad(vals, (0, e_pad - e))
    cols2 = jnp.concatenate([cols_p, cols_p + N_PAD]).reshape(-1, C)
    rows2 = rows_p.reshape(-1, C)
    vals2 = vals_p.reshape(-1, C)

    emb_pad = jnp.pad(embedding.astype(jnp.float32),
                      ((0, N_PAD - N), (0, 0)))
    x0s = jnp.concatenate([emb_pad[:, :HALF], emb_pad[:, HALF:]], axis=0)

    layer_k = _make_kernel(nchunks, final=False)
    final_k = _make_kernel(nchunks, final=True)

    x1s = layer_k(x0s, cols2, rows2, vals2)
    outs = final_k(x1s, cols2, rows2, vals2, x0s, x1s)

    full = jnp.concatenate([outs[:N], outs[N_PAD:N_PAD + N]], axis=1)
    ds3 = N // 3
    return jnp.concatenate(
        [full[:ds3], full[ds3:2 * ds3], full[2 * ds3:]], axis=1)


# speedup vs baseline: 1.4285x; 1.4265x over previous
import functools

import jax
import jax.numpy as jnp
from jax import lax
from jax.experimental import pallas as pl
from jax.experimental.pallas import tpu as pltpu
from jax.experimental.pallas import tpu_sc as plsc

N = 10002
D = 128
HALF = 64
N_PAD = 10240
RPT = 640
C = 128
NTILES = 16
NCORES = 2
VPR = HALF // 16
DEPTH = 4
RDEPTH = 8
UNROLL = 8


def _scale_chunk(gbuf, vb):
    def group(g, carry):
        vv = vb[pl.ds(g * 16, 16)]
        for l in range(0, 16, 2):
            e0 = g * 16 + l
            e1 = e0 + 1
            sv0 = vv[l]
            sv1 = vv[l + 1]
            loads = (
                [gbuf[e0, pl.ds(j * 16, 16)] for j in range(VPR)]
                + [gbuf[e1, pl.ds(j * 16, 16)] for j in range(VPR)]
            )
            prods = ([x * sv0 for x in loads[:VPR]]
                     + [x * sv1 for x in loads[VPR:]])
            for j in range(VPR):
                gbuf[e0, pl.ds(j * 16, 16)] = prods[j]
            for j in range(VPR):
                gbuf[e1, pl.ds(j * 16, 16)] = prods[VPR + j]
        return carry
    lax.fori_loop(0, C // 16, group, 0)


def _body(final, nchunks, *refs):
    if final:
        (x_hbm, cols_hbm, rows_hbm, vals_hbm, x0_hbm, x1_hbm, out_hbm,
         acc, *rest) = refs
    else:
        (x_hbm, cols_hbm, rows_hbm, vals_hbm, out_hbm, acc, *rest) = refs
    cv = rest[0:DEPTH]
    vb = rest[DEPTH:2 * DEPTH]
    gb = rest[2 * DEPTH:3 * DEPTH]
    rv = rest[3 * DEPTH:3 * DEPTH + RDEPTH]
    base = 3 * DEPTH + RDEPTH
    sem_i = rest[base:base + DEPTH]
    sem_g = rest[base + DEPTH:base + 2 * DEPTH]
    sem_s = rest[base + 2 * DEPTH:base + 2 * DEPTH + 2]

    c = lax.axis_index("c")
    s = lax.axis_index("s")

    def zrow(i, carry):
        for j in range(VPR):
            gb[0][i, pl.ds(j * 16, 16)] = jnp.zeros((16,), jnp.float32)
        return carry
    lax.fori_loop(0, C, zrow, 0)
    rbase = s * RPT
    for b in range(RPT // C):
        pltpu.sync_copy(gb[0], acc.at[pl.ds(rbase + b * C, C)])
    plsc.subcore_barrier()

    crow0 = (c * NTILES + s) * nchunks
    erow0 = s * nchunks

    def iter_chunks(i, carry):
        for j in range(UNROLL):
            k = i * UNROLL + j
            s4 = j % DEPTH
            s8 = j % RDEPTH
            p = j % 2
            pltpu.make_async_copy(x_hbm.at[cv[s4]], gb[s4],
                                  sem_g[s4]).wait()
            _scale_chunk(gb[s4], vb[s4])
            if j >= 2:
                pltpu.make_async_copy(gb[(j + 2) % DEPTH],
                                      acc.at[rv[(j + 2) % RDEPTH]],
                                      sem_s[p]).wait()
            else:
                @pl.when(i > 0)
                def _():
                    pltpu.make_async_copy(gb[(j + 2) % DEPTH],
                                          acc.at[rv[(j + 2) % RDEPTH]],
                                          sem_s[p]).wait()
            pltpu.async_copy(gb[s4], acc.at[rv[s8]], sem_s[p], add=True)
            i3s4 = (j + 3) % DEPTH
            i3s8 = (j + 3) % RDEPTH

            def idx3():
                pltpu.async_copy(cols_hbm.at[crow0 + k + 3], cv[i3s4],
                                 sem_i[i3s4])
                pltpu.async_copy(rows_hbm.at[erow0 + k + 3], rv[i3s8],
                                 sem_i[i3s4])
                pltpu.async_copy(vals_hbm.at[erow0 + k + 3], vb[i3s4],
                                 sem_i[i3s4])
            if j < UNROLL - 3:
                idx3()
            else:
                @pl.when(i < nchunks // UNROLL - 1)
                def _():
                    idx3()
            g2s4 = (j + 2) % DEPTH

            def gather2():
                pltpu.make_async_copy(cols_hbm.at[crow0 + k + 2], cv[g2s4],
                                      sem_i[g2s4]).wait()
                pltpu.make_async_copy(rows_hbm.at[erow0 + k + 2],
                                      rv[(j + 2) % RDEPTH],
                                      sem_i[g2s4]).wait()
                pltpu.make_async_copy(vals_hbm.at[erow0 + k + 2], vb[g2s4],
                                      sem_i[g2s4]).wait()
                pltpu.async_copy(x_hbm.at[cv[g2s4]], gb[g2s4], sem_g[g2s4])
            if j < UNROLL - 2:
                gather2()
            else:
                @pl.when(i < nchunks // UNROLL - 1)
                def _():
                    gather2()
        return carry

    for k0 in range(3):
        pltpu.async_copy(cols_hbm.at[crow0 + k0], cv[k0], sem_i[k0])
        pltpu.async_copy(rows_hbm.at[erow0 + k0], rv[k0], sem_i[k0])
        pltpu.async_copy(vals_hbm.at[erow0 + k0], vb[k0], sem_i[k0])
    for k0 in range(2):
        pltpu.make_async_copy(cols_hbm.at[crow0 + k0], cv[k0],
                              sem_i[k0]).wait()
        pltpu.make_async_copy(rows_hbm.at[erow0 + k0], rv[k0],
                              sem_i[k0]).wait()
        pltpu.make_async_copy(vals_hbm.at[erow0 + k0], vb[k0],
                              sem_i[k0]).wait()
        pltpu.async_copy(x_hbm.at[cv[k0]], gb[k0], sem_g[k0])

    lax.fori_loop(0, nchunks // UNROLL, iter_chunks, 0)
    pltpu.make_async_copy(gb[(nchunks - 2) % DEPTH],
                          acc.at[rv[(nchunks - 2) % RDEPTH]],
                          sem_s[(nchunks - 2) % 2]).wait()
    pltpu.make_async_copy(gb[(nchunks - 1) % DEPTH],
                          acc.at[rv[(nchunks - 1) % RDEPTH]],
                          sem_s[(nchunks - 1) % 2]).wait()

    plsc.subcore_barrier()

    obase = c * N_PAD + rbase
    for b in range(RPT // C):
        r0 = rbase + b * C
        o0 = obase + b * C
        if not final:
            pltpu.sync_copy(acc.at[pl.ds(r0, C)], out_hbm.at[pl.ds(o0, C)])
        else:
            pltpu.sync_copy(acc.at[pl.ds(r0, C)], gb[0])
            pltpu.sync_copy(x0_hbm.at[pl.ds(o0, C)], gb[1])
            pltpu.sync_copy(x1_hbm.at[pl.ds(o0, C)], gb[2])

            def crow(i, carry):
                for j in range(VPR):
                    sl = pl.ds(j * 16, 16)
                    gb[0][i, sl] = (
                        gb[0][i, sl] + gb[1][i, sl] + gb[2][i, sl]
                    ) * (1.0 / 3.0)
                return carry
            lax.fori_loop(0, C, crow, 0)
            pltpu.sync_copy(gb[0], out_hbm.at[pl.ds(o0, C)])


def _make_kernel(nchunks, final):
    mesh = plsc.VectorSubcoreMesh(core_axis_name="c", subcore_axis_name="s")
    scratch = [
        pltpu.VMEM_SHARED((N_PAD, HALF), jnp.float32),
    ]
    scratch += [pltpu.VMEM((C,), jnp.int32) for _ in range(DEPTH)]
    scratch += [pltpu.VMEM((C,), jnp.float32) for _ in range(DEPTH)]
    scratch += [pltpu.VMEM((C, HALF), jnp.float32) for _ in range(DEPTH)]
    scratch += [pltpu.VMEM((C,), jnp.int32) for _ in range(RDEPTH)]
    scratch += [pltpu.SemaphoreType.DMA for _ in range(DEPTH)]
    scratch += [pltpu.SemaphoreType.DMA for _ in range(DEPTH)]
    scratch += [pltpu.SemaphoreType.DMA for _ in range(2)]
    return pl.kernel(
        functools.partial(_body, final, nchunks),
        out_type=jax.ShapeDtypeStruct((2 * N_PAD, HALF), jnp.float32),
        mesh=mesh,
        scratch_types=scratch,
        compiler_params=pltpu.CompilerParams(use_tc_tiling_on_sc=False),
    )


def kernel(adjacency_indices, adjacency_values, embedding):
    rows = adjacency_indices[0].astype(jnp.int32)
    cols = adjacency_indices[1].astype(jnp.int32)
    vals = adjacency_values.astype(jnp.float32)
    e = vals.shape[0]
    ept = -(-(e // NTILES) // (UNROLL * C)) * (UNROLL * C)
    nchunks = ept // C
    e_pad = ept * NTILES

    cols_p = jnp.pad(cols, (0, e_pad - e))
    rows_p = jnp.pad(rows, (0, e_pad - e), constant_values=N)
    vals_p = jnp.pad(vals, (0, e_pad - e))
    cols2 = jnp.concatenate([cols_p, cols_p + N_PAD]).reshape(-1, C)
    rows2 = rows_p.reshape(-1, C)
    vals2 = vals_p.reshape(-1, C)

    emb_pad = jnp.pad(embedding.astype(jnp.float32),
                      ((0, N_PAD - N), (0, 0)))
    x0s = jnp.concatenate([emb_pad[:, :HALF], emb_pad[:, HALF:]], axis=0)

    layer_k = _make_kernel(nchunks, final=False)
    final_k = _make_kernel(nchunks, final=True)

    x1s = layer_k(x0s, cols2, rows2, vals2)
    outs = final_k(x1s, cols2, rows2, vals2, x0s, x1s)

    full = jnp.concatenate([outs[:N], outs[N_PAD:N_PAD + N]], axis=1)
    ds3 = N // 3
    return jnp.concatenate(
        [full[:ds3], full[ds3:2 * ds3], full[2 * ds3:]], axis=1)


# speedup vs baseline: 1.4787x; 1.0351x over previous
import functools

import jax
import jax.numpy as jnp
from jax import lax
from jax.experimental import pallas as pl
from jax.experimental.pallas import tpu as pltpu
from jax.experimental.pallas import tpu_sc as plsc

N = 10002
D = 128
HALF = 64
N_PAD = 10240
RPT = 640
C = 128
NTILES = 16
NCORES = 2
VPR = HALF // 16
RING = 8
NSEMS = 4
UNROLL = 8


def _scale_chunk(gbuf, vb):
    def group(g, carry):
        vv = vb[pl.ds(g * 16, 16)]
        for l in range(0, 16, 2):
            e0 = g * 16 + l
            e1 = e0 + 1
            sv0 = vv[l]
            sv1 = vv[l + 1]
            loads = (
                [gbuf[e0, pl.ds(j * 16, 16)] for j in range(VPR)]
                + [gbuf[e1, pl.ds(j * 16, 16)] for j in range(VPR)]
            )
            prods = ([x * sv0 for x in loads[:VPR]]
                     + [x * sv1 for x in loads[VPR:]])
            for j in range(VPR):
                gbuf[e0, pl.ds(j * 16, 16)] = prods[j]
            for j in range(VPR):
                gbuf[e1, pl.ds(j * 16, 16)] = prods[VPR + j]
        return carry
    lax.fori_loop(0, C // 16, group, 0)


def _body(final, nchunks, *refs):
    if final:
        (x_hbm, cols_hbm, rows_hbm, vals_hbm, x0_hbm, x1_hbm, out_hbm,
         acc, *rest) = refs
    else:
        (x_hbm, cols_hbm, rows_hbm, vals_hbm, out_hbm, acc, *rest) = refs
    cv = rest[0:RING]
    vb = rest[RING:2 * RING]
    gb = rest[2 * RING:3 * RING]
    rv = rest[3 * RING:4 * RING]
    base = 4 * RING
    sem_i = rest[base:base + RING]
    sem_g = rest[base + RING:base + 2 * RING]
    sem_s = rest[base + 2 * RING:base + 2 * RING + NSEMS]

    c = lax.axis_index("c")
    s = lax.axis_index("s")

    def zrow(i, carry):
        for j in range(VPR):
            gb[0][i, pl.ds(j * 16, 16)] = jnp.zeros((16,), jnp.float32)
        return carry
    lax.fori_loop(0, C, zrow, 0)
    rbase = s * RPT
    for b in range(RPT // C):
        pltpu.sync_copy(gb[0], acc.at[pl.ds(rbase + b * C, C)])
    plsc.subcore_barrier()

    crow0 = (c * NTILES + s) * nchunks
    erow0 = s * nchunks

    niter = nchunks // UNROLL

    def idx_issue(k, slot):
        pltpu.async_copy(cols_hbm.at[crow0 + k], cv[slot], sem_i[slot])
        pltpu.async_copy(rows_hbm.at[erow0 + k], rv[slot], sem_i[slot])
        pltpu.async_copy(vals_hbm.at[erow0 + k], vb[slot], sem_i[slot])

    def idx_wait(k, slot):
        pltpu.make_async_copy(cols_hbm.at[crow0 + k], cv[slot],
                              sem_i[slot]).wait()
        pltpu.make_async_copy(rows_hbm.at[erow0 + k], rv[slot],
                              sem_i[slot]).wait()
        pltpu.make_async_copy(vals_hbm.at[erow0 + k], vb[slot],
                              sem_i[slot]).wait()

    def iter_chunks(i, carry):
        for j in range(UNROLL):
            k = i * UNROLL + j
            pltpu.make_async_copy(x_hbm.at[cv[j]], gb[j], sem_g[j]).wait()
            _scale_chunk(gb[j], vb[j])
            p = j % NSEMS

            def drain():
                pltpu.make_async_copy(gb[(j + 4) % RING],
                                      acc.at[rv[(j + 4) % RING]],
                                      sem_s[p]).wait()
            if j >= 4:
                drain()
            else:
                @pl.when(i > 0)
                def _():
                    drain()
            pltpu.async_copy(gb[j], acc.at[rv[j]], sem_s[p], add=True)
            if j < UNROLL - 4:
                idx_issue(k + 4, (j + 4) % RING)
            else:
                @pl.when(i < niter - 1)
                def _():
                    idx_issue(k + 4, (j + 4) % RING)
            g3 = (j + 3) % RING

            def gather3():
                idx_wait(k + 3, g3)
                pltpu.async_copy(x_hbm.at[cv[g3]], gb[g3], sem_g[g3])
            if j < UNROLL - 3:
                gather3()
            else:
                @pl.when(i < niter - 1)
                def _():
                    gather3()
        return carry

    for k0 in range(4):
        idx_issue(k0, k0)
    for k0 in range(3):
        idx_wait(k0, k0)
        pltpu.async_copy(x_hbm.at[cv[k0]], gb[k0], sem_g[k0])

    lax.fori_loop(0, niter, iter_chunks, 0)
    for k0 in range(nchunks - 4, nchunks):
        pltpu.make_async_copy(gb[k0 % RING], acc.at[rv[k0 % RING]],
                              sem_s[k0 % NSEMS]).wait()

    plsc.subcore_barrier()

    obase = c * N_PAD + rbase
    for b in range(RPT // C):
        r0 = rbase + b * C
        o0 = obase + b * C
        if not final:
            pltpu.sync_copy(acc.at[pl.ds(r0, C)], out_hbm.at[pl.ds(o0, C)])
        else:
            pltpu.sync_copy(acc.at[pl.ds(r0, C)], gb[0])
            pltpu.sync_copy(x0_hbm.at[pl.ds(o0, C)], gb[1])
            pltpu.sync_copy(x1_hbm.at[pl.ds(o0, C)], gb[2])

            def crow(i, carry):
                for j in range(VPR):
                    sl = pl.ds(j * 16, 16)
                    gb[0][i, sl] = (
                        gb[0][i, sl] + gb[1][i, sl] + gb[2][i, sl]
                    ) * (1.0 / 3.0)
                return carry
            lax.fori_loop(0, C, crow, 0)
            pltpu.sync_copy(gb[0], out_hbm.at[pl.ds(o0, C)])


def _make_kernel(nchunks, final):
    mesh = plsc.VectorSubcoreMesh(core_axis_name="c", subcore_axis_name="s")
    scratch = [
        pltpu.VMEM_SHARED((N_PAD, HALF), jnp.float32),
    ]
    scratch += [pltpu.VMEM((C,), jnp.int32) for _ in range(RING)]
    scratch += [pltpu.VMEM((C,), jnp.float32) for _ in range(RING)]
    scratch += [pltpu.VMEM((C, HALF), jnp.float32) for _ in range(RING)]
    scratch += [pltpu.VMEM((C,), jnp.int32) for _ in range(RING)]
    scratch += [pltpu.SemaphoreType.DMA for _ in range(RING)]
    scratch += [pltpu.SemaphoreType.DMA for _ in range(RING)]
    scratch += [pltpu.SemaphoreType.DMA for _ in range(NSEMS)]
    return pl.kernel(
        functools.partial(_body, final, nchunks),
        out_type=jax.ShapeDtypeStruct((2 * N_PAD, HALF), jnp.float32),
        mesh=mesh,
        scratch_types=scratch,
        compiler_params=pltpu.CompilerParams(use_tc_tiling_on_sc=False),
    )


def kernel(adjacency_indices, adjacency_values, embedding):
    rows = adjacency_indices[0].astype(jnp.int32)
    cols = adjacency_indices[1].astype(jnp.int32)
    vals = adjacency_values.astype(jnp.float32)
    e = vals.shape[0]
    ept = -(-(e // NTILES) // (UNROLL * C)) * (UNROLL * C)
    nchunks = ept // C
    e_pad = ept * NTILES

    cols_p = jnp.pad(cols, (0, e_pad - e))
    rows_p = jnp.pad(rows, (0, e_pad - e), constant_values=N)
    vals_p = jnp.pad(vals, (0, e_pad - e))
    cols2 = jnp.concatenate([cols_p, cols_p + N_PAD]).reshape(-1, C)
    rows2 = rows_p.reshape(-1, C)
    vals2 = vals_p.reshape(-1, C)

    emb_pad = jnp.pad(embedding.astype(jnp.float32),
                      ((0, N_PAD - N), (0, 0)))
    x0s = jnp.concatenate([emb_pad[:, :HALF], emb_pad[:, HALF:]], axis=0)

    layer_k = _make_kernel(nchunks, final=False)
    final_k = _make_kernel(nchunks, final=True)

    x1s = layer_k(x0s, cols2, rows2, vals2)
    outs = final_k(x1s, cols2, rows2, vals2, x0s, x1s)

    full = jnp.concatenate([outs[:N], outs[N_PAD:N_PAD + N]], axis=1)
    ds3 = N // 3
    return jnp.concatenate(
        [full[:ds3], full[ds3:2 * ds3], full[2 * ds3:]], axis=1)


# speedup vs baseline: 2.4752x; 1.6740x over previous
import functools

import jax
import jax.numpy as jnp
from jax import lax
from jax.experimental import pallas as pl
from jax.experimental.pallas import tpu as pltpu
from jax.experimental.pallas import tpu_sc as plsc

N = 10002
D = 128
HALF = 64
N_PAD = 10240
RPT = 640
C = 128
NTILES = 16
VPR = HALF // 16
GRING = 4
RRING = 8
UNROLL = 8


def _scale_chunk(gbuf, vb):
    def group(g, carry):
        vv = vb[pl.ds(g * 16, 16)]
        for l in range(0, 16, 2):
            e0 = g * 16 + l
            e1 = e0 + 1
            sv0 = vv[l]
            sv1 = vv[l + 1]
            loads = (
                [gbuf[e0, pl.ds(j * 16, 16)] for j in range(VPR)]
                + [gbuf[e1, pl.ds(j * 16, 16)] for j in range(VPR)]
            )
            prods = ([x * sv0 for x in loads[:VPR]]
                     + [x * sv1 for x in loads[VPR:]])
            for j in range(VPR):
                gbuf[e0, pl.ds(j * 16, 16)] = prods[j]
            for j in range(VPR):
                gbuf[e1, pl.ds(j * 16, 16)] = prods[VPR + j]
        return carry
    lax.fori_loop(0, C // 16, group, 0)


def _body(nchunks, x_hbm, cols_hbm, rows_hbm, vals_hbm, out_hbm,
          tab, acc, *rest):
    cv = rest[0:GRING]
    vb = rest[GRING:2 * GRING]
    gb = rest[2 * GRING:3 * GRING]
    rv = rest[3 * GRING:3 * GRING + RRING]
    base = 3 * GRING + RRING
    sem_i = rest[base:base + GRING]
    sem_g = rest[base + GRING:base + 2 * GRING]
    sem_s = rest[base + 2 * GRING:base + 2 * GRING + 2]

    c = lax.axis_index("c")
    s = lax.axis_index("s")
    coff = c * N_PAD
    rbase = s * RPT
    erow0 = s * nchunks
    niter = nchunks // UNROLL

    def zero_gb0():
        def zrow(i, carry):
            for j in range(VPR):
                gb[0][i, pl.ds(j * 16, 16)] = jnp.zeros((16,), jnp.float32)
            return carry
        lax.fori_loop(0, C, zrow, 0)

    zero_gb0()
    for b in range(RPT // C):
        r0 = rbase + b * C
        pltpu.sync_copy(x_hbm.at[pl.ds(coff + r0, C)], tab.at[pl.ds(r0, C)])
        pltpu.sync_copy(gb[0], acc.at[pl.ds(r0, C)])
    plsc.subcore_barrier()

    def run_layer(src, dst):
        def idx_go(k, g4, r8):
            pltpu.async_copy(cols_hbm.at[erow0 + k], cv[g4], sem_i[g4])
            pltpu.async_copy(rows_hbm.at[erow0 + k], rv[r8], sem_i[g4])
            pltpu.async_copy(vals_hbm.at[erow0 + k], vb[g4], sem_i[g4])

        def idx_wait(k, g4, r8):
            pltpu.make_async_copy(cols_hbm.at[erow0 + k], cv[g4],
                                  sem_i[g4]).wait()
            pltpu.make_async_copy(rows_hbm.at[erow0 + k], rv[r8],
                                  sem_i[g4]).wait()
            pltpu.make_async_copy(vals_hbm.at[erow0 + k], vb[g4],
                                  sem_i[g4]).wait()

        def iter_chunks(i, carry):
            for j in range(UNROLL):
                k = i * UNROLL + j
                g4 = j % GRING
                r8 = j % RRING
                p = j % 2
                pltpu.make_async_copy(src.at[cv[g4]], gb[g4],
                                      sem_g[g4]).wait()
                _scale_chunk(gb[g4], vb[g4])

                def drain():
                    pltpu.make_async_copy(gb[(j + 2) % GRING],
                                          dst.at[rv[(j + 2) % RRING]],
                                          sem_s[p]).wait()
                if j >= 2:
                    drain()
                else:
                    @pl.when(i > 0)
                    def _():
                        drain()
                pltpu.async_copy(gb[g4], dst.at[rv[r8]], sem_s[p], add=True)

                if j < UNROLL - 3:
                    idx_go(k + 3, (j + 3) % GRING, (j + 3) % RRING)
                else:
                    @pl.when(i < niter - 1)
                    def _():
                        idx_go(k + 3, (j + 3) % GRING, (j + 3) % RRING)

                def gather2():
                    g2 = (j + 2) % GRING
                    idx_wait(k + 2, g2, (j + 2) % RRING)
                    pltpu.async_copy(src.at[cv[g2]], gb[g2], sem_g[g2])
                if j < UNROLL - 2:
                    gather2()
                else:
                    @pl.when(i < niter - 1)
                    def _():
                        gather2()
            return carry

        for k0 in range(3):
            idx_go(k0, k0, k0)
        for k0 in range(2):
            idx_wait(k0, k0, k0)
            pltpu.async_copy(src.at[cv[k0]], gb[k0], sem_g[k0])
        lax.fori_loop(0, niter, iter_chunks, 0)
        for k0 in range(nchunks - 2, nchunks):
            pltpu.make_async_copy(gb[k0 % GRING], dst.at[rv[k0 % RRING]],
                                  sem_s[k0 % 2]).wait()

    run_layer(tab, acc)
    plsc.subcore_barrier()

    zero_gb0()
    for b in range(RPT // C):
        pltpu.sync_copy(gb[0], tab.at[pl.ds(rbase + b * C, C)])
    plsc.subcore_barrier()

    run_layer(acc, tab)
    plsc.subcore_barrier()

    for b in range(RPT // C):
        r0 = rbase + b * C
        o0 = coff + r0
        pltpu.sync_copy(x_hbm.at[pl.ds(o0, C)], gb[0])
        pltpu.sync_copy(acc.at[pl.ds(r0, C)], gb[1])
        pltpu.sync_copy(tab.at[pl.ds(r0, C)], gb[2])

        def crow(i, carry):
            for j in range(VPR):
                sl = pl.ds(j * 16, 16)
                gb[0][i, sl] = (
                    gb[0][i, sl] + gb[1][i, sl] + gb[2][i, sl]
                ) * (1.0 / 3.0)
            return carry
        lax.fori_loop(0, C, crow, 0)
        pltpu.sync_copy(gb[0], out_hbm.at[pl.ds(o0, C)])


def _make_kernel(nchunks):
    mesh = plsc.VectorSubcoreMesh(core_axis_name="c", subcore_axis_name="s")
    scratch = [
        pltpu.VMEM_SHARED((N_PAD, HALF), jnp.float32),
        pltpu.VMEM_SHARED((N_PAD, HALF), jnp.float32),
    ]
    scratch += [pltpu.VMEM((C,), jnp.int32) for _ in range(GRING)]
    scratch += [pltpu.VMEM((C,), jnp.float32) for _ in range(GRING)]
    scratch += [pltpu.VMEM((C, HALF), jnp.float32) for _ in range(GRING)]
    scratch += [pltpu.VMEM((C,), jnp.int32) for _ in range(RRING)]
    scratch += [pltpu.SemaphoreType.DMA for _ in range(GRING)]
    scratch += [pltpu.SemaphoreType.DMA for _ in range(GRING)]
    scratch += [pltpu.SemaphoreType.DMA for _ in range(2)]
    return pl.kernel(
        functools.partial(_body, nchunks),
        out_type=jax.ShapeDtypeStruct((2 * N_PAD, HALF), jnp.float32),
        mesh=mesh,
        scratch_types=scratch,
        compiler_params=pltpu.CompilerParams(use_tc_tiling_on_sc=False),
    )


def kernel(adjacency_indices, adjacency_values, embedding):
    rows = adjacency_indices[0].astype(jnp.int32)
    cols = adjacency_indices[1].astype(jnp.int32)
    vals = adjacency_values.astype(jnp.float32)
    e = vals.shape[0]
    ept = -(-(e // NTILES) // (UNROLL * C)) * (UNROLL * C)
    nchunks = ept // C
    e_pad = ept * NTILES

    cols2 = jnp.pad(cols, (0, e_pad - e)).reshape(-1, C)
    rows2 = jnp.pad(rows, (0, e_pad - e),
                    constant_values=N).reshape(-1, C)
    vals2 = jnp.pad(vals, (0, e_pad - e)).reshape(-1, C)

    emb_pad = jnp.pad(embedding.astype(jnp.float32),
                      ((0, N_PAD - N), (0, 0)))
    x0s = jnp.concatenate([emb_pad[:, :HALF], emb_pad[:, HALF:]], axis=0)

    outs = _make_kernel(nchunks)(x0s, cols2, rows2, vals2)

    full = jnp.concatenate([outs[:N], outs[N_PAD:N_PAD + N]], axis=1)
    ds3 = N // 3
    return jnp.concatenate(
        [full[:ds3], full[ds3:2 * ds3], full[2 * ds3:]], axis=1)
